# baseline (speedup 1.0000x reference)
import functools

import jax
import jax.numpy as jnp
from jax import lax
from jax.experimental import pallas as pl
from jax.experimental.pallas import tpu as pltpu
from jax.experimental.pallas import tpu_sc as plsc

N = 4096
G = 64
F = 64
NBR = 64
K = 12
NCONV = 3
LATENT = 128
E = N * K
BIG = 1e12
INF = 1e30
VALID_T = 1e5

R = 128
CCH = 256
BN = 256
EPS_BN = 1e-5

_SC_NC, _SC_NS = 2, 16
_SC_NW = _SC_NC * _SC_NS


def _sc_gather(table, idx):
    V, D = table.shape
    B = idx.shape[0]
    bpw = B // _SC_NW
    ch = min(bpw, 512)
    rounds = bpw // ch
    mesh = plsc.VectorSubcoreMesh(
        core_axis_name="c", subcore_axis_name="s",
        num_cores=_SC_NC, num_subcores=_SC_NS)

    @functools.partial(
        pl.kernel, mesh=mesh,
        out_type=jax.ShapeDtypeStruct((B, D), jnp.float32),
        scratch_types=[
            pltpu.VMEM((ch,), jnp.int32),
            pltpu.VMEM((ch, D), jnp.float32),
            pltpu.SemaphoreType.DMA,
        ])
    def gk(table_hbm, idx_hbm, out_hbm, idx_v, rows_v, sem):
        wid = lax.axis_index("s") * _SC_NC + lax.axis_index("c")
        base = wid * bpw
        for r in range(rounds):
            b = base + r * ch
            pltpu.sync_copy(idx_hbm.at[pl.ds(b, ch)], idx_v)
            pltpu.async_copy(table_hbm.at[idx_v], rows_v, sem).wait()
            pltpu.sync_copy(rows_v, out_hbm.at[pl.ds(b, ch)])

    return gk(table, idx)


def _cart_kernel(fracsT_ref, batchT_ref, bcol_ref, lat_ref, ltri_ref,
                 out_ref, rs_ref, re_ref):
    bt = batchT_ref[...]
    lat = lat_ref[...]
    oh = (lax.broadcasted_iota(jnp.int32, (G, 1), 0) == bt
          ).astype(jnp.float32)
    acc = jnp.dot(lat, oh, precision=lax.Precision.HIGHEST,
                  preferred_element_type=jnp.float32)
    f0 = fracsT_ref[0:1, :]
    f1 = fracsT_ref[1:2, :]
    f2 = fracsT_ref[2:3, :]
    rows = []
    for d in range(3):
        rows.append(f0 * acc[d:d + 1, :] + f1 * acc[3 + d:4 + d, :]
                    + f2 * acc[6 + d:7 + d, :])
    si = lax.broadcasted_iota(jnp.int32, (8, N), 0)
    out = jnp.where(si == 0, rows[0],
                    jnp.where(si == 1, rows[1],
                              jnp.where(si == 2, rows[2], 0.0)))
    out_ref[...] = out
    bcol = bcol_ref[...]
    ohT = (bcol == lax.broadcasted_iota(jnp.int32, (1, G), 1)
           ).astype(jnp.float32)
    ones = jnp.ones((N, 1), jnp.float32)
    hi = lax.Precision.HIGHEST
    cnt_col = jnp.dot(oh, ones, precision=hi,
                      preferred_element_type=jnp.float32)
    seg_s = jnp.dot(ltri_ref[...], cnt_col, precision=hi,
                    preferred_element_type=jnp.float32)
    seg_e = seg_s + cnt_col
    rs = jnp.dot(ohT, seg_s, precision=hi,
                 preferred_element_type=jnp.float32)
    re = jnp.dot(ohT, seg_e, precision=hi,
                 preferred_element_type=jnp.float32)
    rs_ref[...] = rs.astype(jnp.int32)
    re_ref[...] = re.astype(jnp.int32)


def _topk_kernel(rs_p, re_p, cartT_ref, batchT_ref, cartn_ref, bcol_ref,
                 rs_ref, re_ref, idx_ref, dist_ref):
    pid = pl.program_id(0)
    r0 = pid * R
    lo = rs_p[r0]
    hi = re_p[r0 + R - 1]
    c_lo = lo // CCH
    c_hi = (hi + CCH - 1) // CCH

    xb = cartn_ref[...]
    x0 = xb[:, 0:1]
    x1 = xb[:, 1:2]
    x2 = xb[:, 2:3]
    brow = bcol_ref[...]
    rowid = r0 + lax.broadcasted_iota(jnp.int32, (R, 1), 0)

    def chunk_body(c, carry):
        cv, ci = carry
        cb = c * CCH
        btc = batchT_ref[:, pl.ds(cb, CCH)]
        colid = cb + lax.broadcasted_iota(jnp.int32, (1, CCH), 1)
        d0 = x0 - cartT_ref[0:1, pl.ds(cb, CCH)]
        d1 = x1 - cartT_ref[1:2, pl.ds(cb, CCH)]
        d2 = x2 - cartT_ref[2:3, pl.ds(cb, CCH)]
        dist = d0 * d0 + d1 * d1 + d2 * d2
        ok = (btc == brow) & (colid != rowid)
        dist = jnp.where(ok, dist, BIG)
        idxf = jnp.broadcast_to(colid.astype(jnp.float32), (R, CCH))
        av = jnp.concatenate([cv, dist], axis=1)
        ai = jnp.concatenate([ci, idxf], axis=1)
        lane = lax.broadcasted_iota(jnp.int32, (R, 128), 1)
        nv = jnp.full((R, 128), INF, jnp.float32)
        ni = jnp.full((R, 128), 1e9, jnp.float32)
        for t in range(K):
            m = jnp.min(av, axis=1, keepdims=True)
            ismin = av == m
            sel = jnp.min(jnp.where(ismin, ai, 1e9), axis=1, keepdims=True)
            hit = ismin & (ai == sel)
            nv = jnp.where(lane == t, m, nv)
            ni = jnp.where(lane == t, sel, ni)
            av = jnp.where(hit, INF, av)
        return nv, ni

    cv0 = jnp.full((R, 128), INF, jnp.float32)
    ci0 = jnp.full((R, 128), 1e9, jnp.float32)
    cv, ci = lax.fori_loop(c_lo, c_hi, chunk_body, (cv0, ci0))

    s = rs_ref[...]
    e = re_ref[...]
    cnt = e - s - 1
    tl = lax.broadcasted_iota(jnp.int32, (R, 16), 1)
    vals = cv[:, :16]
    idxs = ci[:, :16]
    j = tl - cnt
    fill = jnp.where(j < s, j,
                     jnp.where(j == s, rowid, e + (j - s) - 1))
    use_fill = tl >= cnt
    fidx = jnp.where(use_fill, fill.astype(jnp.float32), idxs)
    fval = jnp.where(use_fill, jnp.float32(BIG), vals)
    idx_ref[...] = fidx
    dist_ref[...] = jnp.sqrt(fval + 1e-8)


def _convA_kernel(asrc_ref, atom_ref, dist_ref, offs_ref, coef_ref,
                  w_ref, b_ref, z_ref, st_ref):
    i = pl.program_id(0)
    atom = atom_ref[...][:, :F]
    w = w_ref[...]
    b = b_ref[...]
    offs = offs_ref[...]
    coef = coef_ref[...]
    asrc = asrc_ref[...][:, :, :F].reshape(K * BN, F)
    adst = jnp.concatenate([atom] * K, axis=0)
    nbrs = []
    for t in range(K):
        dcol = dist_ref[:, t:t + 1]
        valid = (dcol < VALID_T).astype(jnp.float32)
        dlt = dcol - offs
        nbrs.append(jnp.exp(coef * dlt * dlt) * valid)
    nbr = jnp.concatenate(nbrs, axis=0)
    tot = jnp.concatenate([asrc, adst, nbr], axis=1)
    z = jnp.dot(tot, w, preferred_element_type=jnp.float32) + b
    z_ref[...] = z.reshape(K, BN, 2 * F)
    ssum = jnp.sum(z, axis=0, keepdims=True)
    ssq = jnp.sum(z * z, axis=0, keepdims=True)
    sub = lax.broadcasted_iota(jnp.int32, (8, 2 * F), 0)
    contrib = (jnp.where(sub == 0, ssum, 0.0)
               + jnp.where(sub == 1, ssq, 0.0))

    @pl.when(i == 0)
    def _():
        st_ref[...] = jnp.zeros((8, 2 * F), jnp.float32)

    st_ref[...] += contrib


def _softplus(x):
    return jnp.maximum(x, 0.0) + jnp.log1p(jnp.exp(-jnp.abs(x)))


def _sigmoid(x):
    return 1.0 / (1.0 + jnp.exp(-x))


def _convB_kernel(z_ref, st_ref, g_ref, b_ref, dist_ref, upd_ref, st2_ref):
    i = pl.program_id(0)
    st = st_ref[...]
    mean = st[0:1] * (1.0 / E)
    var = st[1:2] * (1.0 / E) - mean * mean
    mult = g_ref[...] * lax.rsqrt(var + EPS_BN)
    add = b_ref[...] - mean * mult
    upd = jnp.zeros((BN, F), jnp.float32)
    for t in range(K):
        z = z_ref[t] * mult + add
        filt = z[:, :F]
        core = z[:, F:]
        dcol = dist_ref[:, t:t + 1]
        valid = (dcol < VALID_T).astype(jnp.float32)
        upd = upd + _sigmoid(filt) * _softplus(core) * valid
    upd_ref[...] = upd
    s0 = jnp.sum(upd, axis=0, keepdims=True)
    s1 = jnp.sum(upd * upd, axis=0, keepdims=True)
    sub = lax.broadcasted_iota(jnp.int32, (8, F), 0)
    contrib = (jnp.where(sub == 0, s0, 0.0)
               + jnp.where(sub == 1, s1, 0.0))

    @pl.when(i == 0)
    def _():
        st2_ref[...] = jnp.zeros((8, F), jnp.float32)

    st2_ref[...] += contrib


def _convC_kernel(atom_ref, upd_ref, st2_ref, g_ref, b_ref, out_ref):
    st = st2_ref[...]
    mean = st[0:1] * (1.0 / N)
    var = st[1:2] * (1.0 / N) - mean * mean
    mult = g_ref[...] * lax.rsqrt(var + EPS_BN)
    add = b_ref[...] - mean * mult
    x = atom_ref[...][:, :F] + upd_ref[...] * mult + add
    out_ref[:, :F] = _softplus(x)
    out_ref[:, F:] = jnp.zeros((N, 128 - F), jnp.float32)


def _final_kernel(atom_ref, batchT_ref, lat_ref, w1aT_ref, w1bT_ref,
                  b1_ref, w2T_ref, b2_ref, mu_ref, lv_ref):
    bt = batchT_ref[...]
    gi = lax.broadcasted_iota(jnp.int32, (G, N), 0)
    oh = (gi == bt).astype(jnp.float32)
    crys = jnp.dot(oh, atom_ref[...][:, :F],
                   preferred_element_type=jnp.float32)
    cnt = jnp.sum(oh, axis=1, keepdims=True)
    crys = crys / jnp.maximum(cnt, 1.0)
    h = (jnp.dot(crys, w1aT_ref[...], preferred_element_type=jnp.float32)
         + jnp.dot(lat_ref[...], w1bT_ref[...],
                   preferred_element_type=jnp.float32)
         + b1_ref[...])
    h = h * _sigmoid(h)
    out = (jnp.dot(h, w2T_ref[...], preferred_element_type=jnp.float32)
           + b2_ref[...])
    mu_ref[...] = out[:, :LATENT]
    lv_ref[...] = out[:, LATENT:]


def kernel(lattice, fracs, species, batch_indices, emb, fc_w, fc_b,
           bn1_g, bn1_b, bn2_g, bn2_b, w1, b1, w2, b2):
    f32 = jnp.float32
    batch = batch_indices.astype(jnp.int32)
    batchT = batch.reshape(1, N)
    batch_col = batch.reshape(N, 1)
    fracsT8 = jnp.zeros((8, N), f32).at[:3].set(fracs.T)
    lat9 = lattice.reshape(G, 9)
    latT16 = jnp.zeros((16, G), f32).at[:9].set(lat9.T)

    ltri = (jnp.arange(G)[:, None] > jnp.arange(G)[None, :]).astype(f32)
    cartT, rs_col, re_col = pl.pallas_call(
        _cart_kernel,
        out_shape=[jax.ShapeDtypeStruct((8, N), f32),
                   jax.ShapeDtypeStruct((N, 1), jnp.int32),
                   jax.ShapeDtypeStruct((N, 1), jnp.int32)],
    )(fracsT8, batchT, batch_col, latT16, ltri)

    cart_nt = jnp.zeros((N, 4), f32).at[:, :3].set(cartT[:3].T)
    row_s = rs_col.reshape(N)
    row_e = re_col.reshape(N)

    grid_spec = pltpu.PrefetchScalarGridSpec(
        num_scalar_prefetch=2,
        grid=(N // R,),
        in_specs=[
            pl.BlockSpec((8, N), lambda i, *_: (0, 0)),
            pl.BlockSpec((1, N), lambda i, *_: (0, 0)),
            pl.BlockSpec((R, 4), lambda i, *_: (i, 0)),
            pl.BlockSpec((R, 1), lambda i, *_: (i, 0)),
            pl.BlockSpec((R, 1), lambda i, *_: (i, 0)),
            pl.BlockSpec((R, 1), lambda i, *_: (i, 0)),
        ],
        out_specs=[
            pl.BlockSpec((R, 16), lambda i, *_: (i, 0)),
            pl.BlockSpec((R, 16), lambda i, *_: (i, 0)),
        ],
    )
    idx16, dist16 = pl.pallas_call(
        _topk_kernel,
        grid_spec=grid_spec,
        out_shape=[jax.ShapeDtypeStruct((N, 16), f32),
                   jax.ShapeDtypeStruct((N, 16), f32)],
    )(row_s, row_e, cartT, batchT, cart_nt, batch_col, rs_col, re_col)

    src_rm = idx16[:, :K].T.reshape(E).astype(jnp.int32)

    emb_pad = jnp.zeros((emb.shape[0], 128), f32).at[:, :F].set(emb)
    atom = _sc_gather(emb_pad, species.astype(jnp.int32))

    offs = jnp.linspace(0.0, 8.0, NBR).astype(f32).reshape(1, NBR)
    coef = (-0.5 / (offs[0, 1] - offs[0, 0]) ** 2).reshape(1, 1)

    for i in range(NCONV):
        asrc = _sc_gather(atom, src_rm).reshape(K, N, 128)
        fcwT = fc_w[i].T
        fcb = fc_b[i].reshape(1, 2 * F)
        z3, st = pl.pallas_call(
            _convA_kernel,
            grid=(N // BN,),
            in_specs=[
                pl.BlockSpec((K, BN, 128), lambda n: (0, n, 0)),
                pl.BlockSpec((BN, 128), lambda n: (n, 0)),
                pl.BlockSpec((BN, 16), lambda n: (n, 0)),
                pl.BlockSpec((1, NBR), lambda n: (0, 0)),
                pl.BlockSpec((1, 1), lambda n: (0, 0)),
                pl.BlockSpec((3 * F, 2 * F), lambda n: (0, 0)),
                pl.BlockSpec((1, 2 * F), lambda n: (0, 0)),
            ],
            out_specs=[
                pl.BlockSpec((K, BN, 2 * F), lambda n: (0, n, 0)),
                pl.BlockSpec((8, 2 * F), lambda n: (0, 0)),
            ],
            out_shape=[jax.ShapeDtypeStruct((K, N, 2 * F), f32),
                       jax.ShapeDtypeStruct((8, 2 * F), f32)],
        )(asrc, atom, dist16, offs, coef, fcwT, fcb)

        upd, st2 = pl.pallas_call(
            _convB_kernel,
            grid=(N // BN,),
            in_specs=[
                pl.BlockSpec((K, BN, 2 * F), lambda n: (0, n, 0)),
                pl.BlockSpec((8, 2 * F), lambda n: (0, 0)),
                pl.BlockSpec((1, 2 * F), lambda n: (0, 0)),
                pl.BlockSpec((1, 2 * F), lambda n: (0, 0)),
                pl.BlockSpec((BN, 16), lambda n: (n, 0)),
            ],
            out_specs=[
                pl.BlockSpec((BN, F), lambda n: (n, 0)),
                pl.BlockSpec((8, F), lambda n: (0, 0)),
            ],
            out_shape=[jax.ShapeDtypeStruct((N, F), f32),
                       jax.ShapeDtypeStruct((8, F), f32)],
        )(z3, st, bn1_g[i].reshape(1, 2 * F), bn1_b[i].reshape(1, 2 * F),
          dist16)

        atom = pl.pallas_call(
            _convC_kernel,
            out_shape=jax.ShapeDtypeStruct((N, 128), f32),
        )(atom, upd, st2, bn2_g[i].reshape(1, F), bn2_b[i].reshape(1, F))

    lat16 = jnp.zeros((G, 16), f32).at[:, :9].set(lat9)
    w1aT = w1[:, :F].T
    w1bT = jnp.zeros((16, 128), f32).at[:9].set(w1[:, F:].T)
    mu, lv = pl.pallas_call(
        _final_kernel,
        out_shape=[jax.ShapeDtypeStruct((G, LATENT), f32),
                   jax.ShapeDtypeStruct((G, LATENT), f32)],
    )(atom, batchT, lat16, w1aT, w1bT, b1.reshape(1, 128),
      w2.T, b2.reshape(1, 2 * LATENT))
    return mu, lv


# speedup vs baseline: 1.0155x; 1.0155x over previous
import functools

import jax
import jax.numpy as jnp
from jax import lax
from jax.experimental import pallas as pl
from jax.experimental.pallas import tpu as pltpu
from jax.experimental.pallas import tpu_sc as plsc

N = 4096
G = 64
F = 64
NBR = 64
K = 12
NCONV = 3
LATENT = 128
E = N * K
BIG = 1e12
INF = 1e30
VALID_T = 1e5

R = 128
CCH = 256
BN = 256
EPS_BN = 1e-5

_SC_NC, _SC_NS = 2, 16
_SC_NW = _SC_NC * _SC_NS


def _sc_gather(table, idx):
    V, D = table.shape
    B = idx.shape[0]
    bpw = B // _SC_NW
    ch = min(bpw, 512)
    rounds = bpw // ch
    mesh = plsc.VectorSubcoreMesh(
        core_axis_name="c", subcore_axis_name="s",
        num_cores=_SC_NC, num_subcores=_SC_NS)

    @functools.partial(
        pl.kernel, mesh=mesh,
        out_type=jax.ShapeDtypeStruct((B, D), table.dtype),
        scratch_types=[
            pltpu.VMEM((ch,), jnp.int32),
            pltpu.VMEM((ch, D), table.dtype),
            pltpu.SemaphoreType.DMA,
        ])
    def gk(table_hbm, idx_hbm, out_hbm, idx_v, rows_v, sem):
        wid = lax.axis_index("s") * _SC_NC + lax.axis_index("c")
        base = wid * bpw
        for r in range(rounds):
            b = base + r * ch
            pltpu.sync_copy(idx_hbm.at[pl.ds(b, ch)], idx_v)
            pltpu.async_copy(table_hbm.at[idx_v], rows_v, sem).wait()
            pltpu.sync_copy(rows_v, out_hbm.at[pl.ds(b, ch)])

    return gk(table, idx)


def _cart_kernel(fracsT_ref, batchT_ref, bcol_ref, lat_ref, ltri_ref,
                 out_ref, rs_ref, re_ref):
    bt = batchT_ref[...]
    lat = lat_ref[...]
    oh = (lax.broadcasted_iota(jnp.int32, (G, 1), 0) == bt
          ).astype(jnp.float32)
    acc = jnp.dot(lat, oh, precision=lax.Precision.HIGHEST,
                  preferred_element_type=jnp.float32)
    f0 = fracsT_ref[0:1, :]
    f1 = fracsT_ref[1:2, :]
    f2 = fracsT_ref[2:3, :]
    rows = []
    for d in range(3):
        rows.append(f0 * acc[d:d + 1, :] + f1 * acc[3 + d:4 + d, :]
                    + f2 * acc[6 + d:7 + d, :])
    si = lax.broadcasted_iota(jnp.int32, (8, N), 0)
    out = jnp.where(si == 0, rows[0],
                    jnp.where(si == 1, rows[1],
                              jnp.where(si == 2, rows[2], 0.0)))
    out_ref[...] = out
    bcol = bcol_ref[...]
    ohT = (bcol == lax.broadcasted_iota(jnp.int32, (1, G), 1)
           ).astype(jnp.float32)
    ones = jnp.ones((N, 1), jnp.float32)
    hi = lax.Precision.HIGHEST
    cnt_col = jnp.dot(oh, ones, precision=hi,
                      preferred_element_type=jnp.float32)
    seg_s = jnp.dot(ltri_ref[...], cnt_col, precision=hi,
                    preferred_element_type=jnp.float32)
    seg_e = seg_s + cnt_col
    rs = jnp.dot(ohT, seg_s, precision=hi,
                 preferred_element_type=jnp.float32)
    re = jnp.dot(ohT, seg_e, precision=hi,
                 preferred_element_type=jnp.float32)
    rs_ref[...] = rs.astype(jnp.int32)
    re_ref[...] = re.astype(jnp.int32)


def _topk_kernel(rs_p, re_p, cartT_ref, batchT_ref, cartn_ref, bcol_ref,
                 rs_ref, re_ref, idx_ref, dist_ref):
    pid = pl.program_id(0)
    r0 = pid * R
    lo = rs_p[r0]
    hi = re_p[r0 + R - 1]
    c_lo = lo // CCH
    c_hi = (hi + CCH - 1) // CCH

    xb = cartn_ref[...]
    x0 = xb[:, 0:1]
    x1 = xb[:, 1:2]
    x2 = xb[:, 2:3]
    brow = bcol_ref[...]
    rowid = r0 + lax.broadcasted_iota(jnp.int32, (R, 1), 0)

    def chunk_body(c, carry):
        cv, ci = carry
        cb = c * CCH
        btc = batchT_ref[:, pl.ds(cb, CCH)]
        colid = cb + lax.broadcasted_iota(jnp.int32, (1, CCH), 1)
        d0 = x0 - cartT_ref[0:1, pl.ds(cb, CCH)]
        d1 = x1 - cartT_ref[1:2, pl.ds(cb, CCH)]
        d2 = x2 - cartT_ref[2:3, pl.ds(cb, CCH)]
        dist = d0 * d0 + d1 * d1 + d2 * d2
        ok = (btc == brow) & (colid != rowid)
        dist = jnp.where(ok, dist, BIG)
        idxf = jnp.broadcast_to(colid.astype(jnp.float32), (R, CCH))
        av = jnp.concatenate([cv, dist], axis=1)
        ai = jnp.concatenate([ci, idxf], axis=1)
        lane = lax.broadcasted_iota(jnp.int32, (R, 128), 1)
        nv = jnp.full((R, 128), INF, jnp.float32)
        ni = jnp.full((R, 128), 1e9, jnp.float32)
        for t in range(K):
            m = jnp.min(av, axis=1, keepdims=True)
            ismin = av == m
            sel = jnp.min(jnp.where(ismin, ai, 1e9), axis=1, keepdims=True)
            hit = ismin & (ai == sel)
            nv = jnp.where(lane == t, m, nv)
            ni = jnp.where(lane == t, sel, ni)
            av = jnp.where(hit, INF, av)
        return nv, ni

    cv0 = jnp.full((R, 128), INF, jnp.float32)
    ci0 = jnp.full((R, 128), 1e9, jnp.float32)
    cv, ci = lax.fori_loop(c_lo, c_hi, chunk_body, (cv0, ci0))

    s = rs_ref[...]
    e = re_ref[...]
    cnt = e - s - 1
    tl = lax.broadcasted_iota(jnp.int32, (R, 16), 1)
    vals = cv[:, :16]
    idxs = ci[:, :16]
    j = tl - cnt
    fill = jnp.where(j < s, j,
                     jnp.where(j == s, rowid, e + (j - s) - 1))
    use_fill = tl >= cnt
    fidx = jnp.where(use_fill, fill.astype(jnp.float32), idxs)
    fval = jnp.where(use_fill, jnp.float32(BIG), vals)
    idx_ref[...] = fidx
    dist_ref[...] = jnp.sqrt(fval + 1e-8)


def _convA_kernel(asrc_ref, atom_ref, dist_ref, offs_ref, coef_ref,
                  w_ref, b_ref, z_ref, st_ref):
    i = pl.program_id(0)
    atom = atom_ref[...][:, :F].astype(jnp.bfloat16)
    w = w_ref[...]
    b = b_ref[...]
    offs = offs_ref[...]
    coef = coef_ref[...]
    asrc = asrc_ref[...][:, :, :F].astype(jnp.bfloat16).reshape(K * BN, F)
    adst = jnp.concatenate([atom] * K, axis=0)
    nbrs = []
    for t in range(K):
        dcol = dist_ref[:, t:t + 1]
        valid = (dcol < VALID_T).astype(jnp.float32)
        dlt = dcol - offs
        nbrs.append((jnp.exp(coef * dlt * dlt) * valid
                     ).astype(jnp.bfloat16))
    nbr = jnp.concatenate(nbrs, axis=0)
    tot = jnp.concatenate([asrc, adst, nbr], axis=1)
    z = jnp.dot(tot, w.astype(jnp.bfloat16),
                preferred_element_type=jnp.float32) + b
    z_ref[...] = z.astype(jnp.bfloat16).reshape(K, BN, 2 * F)
    ssum = jnp.sum(z, axis=0, keepdims=True)
    ssq = jnp.sum(z * z, axis=0, keepdims=True)
    sub = lax.broadcasted_iota(jnp.int32, (8, 2 * F), 0)
    contrib = (jnp.where(sub == 0, ssum, 0.0)
               + jnp.where(sub == 1, ssq, 0.0))

    @pl.when(i == 0)
    def _():
        st_ref[...] = jnp.zeros((8, 2 * F), jnp.float32)

    st_ref[...] += contrib


def _softplus(x):
    return jnp.maximum(x, 0.0) + jnp.log1p(jnp.exp(-jnp.abs(x)))


def _sigmoid(x):
    return 1.0 / (1.0 + jnp.exp(-x))


def _convB_kernel(z_ref, st_ref, g_ref, b_ref, dist_ref, upd_ref, st2_ref):
    i = pl.program_id(0)
    st = st_ref[...]
    mean = st[0:1] * (1.0 / E)
    var = st[1:2] * (1.0 / E) - mean * mean
    mult = g_ref[...] * lax.rsqrt(var + EPS_BN)
    add = b_ref[...] - mean * mult
    upd = jnp.zeros((BN, F), jnp.float32)
    for t in range(K):
        z = z_ref[t].astype(jnp.float32) * mult + add
        filt = z[:, :F]
        core = z[:, F:]
        dcol = dist_ref[:, t:t + 1]
        valid = (dcol < VALID_T).astype(jnp.float32)
        upd = upd + _sigmoid(filt) * _softplus(core) * valid
    upd_ref[...] = upd
    s0 = jnp.sum(upd, axis=0, keepdims=True)
    s1 = jnp.sum(upd * upd, axis=0, keepdims=True)
    sub = lax.broadcasted_iota(jnp.int32, (8, F), 0)
    contrib = (jnp.where(sub == 0, s0, 0.0)
               + jnp.where(sub == 1, s1, 0.0))

    @pl.when(i == 0)
    def _():
        st2_ref[...] = jnp.zeros((8, F), jnp.float32)

    st2_ref[...] += contrib


def _convC_kernel(atom_ref, upd_ref, st2_ref, g_ref, b_ref, out_ref):
    st = st2_ref[...]
    mean = st[0:1] * (1.0 / N)
    var = st[1:2] * (1.0 / N) - mean * mean
    mult = g_ref[...] * lax.rsqrt(var + EPS_BN)
    add = b_ref[...] - mean * mult
    x = atom_ref[...][:, :F] + upd_ref[...] * mult + add
    out_ref[:, :F] = _softplus(x)
    out_ref[:, F:] = jnp.zeros((N, 128 - F), jnp.float32)


def _final_kernel(atom_ref, batchT_ref, lat_ref, w1aT_ref, w1bT_ref,
                  b1_ref, w2T_ref, b2_ref, mu_ref, lv_ref):
    bt = batchT_ref[...]
    gi = lax.broadcasted_iota(jnp.int32, (G, N), 0)
    oh = (gi == bt).astype(jnp.float32)
    crys = jnp.dot(oh, atom_ref[...][:, :F],
                   preferred_element_type=jnp.float32)
    cnt = jnp.sum(oh, axis=1, keepdims=True)
    crys = crys / jnp.maximum(cnt, 1.0)
    h = (jnp.dot(crys, w1aT_ref[...], preferred_element_type=jnp.float32)
         + jnp.dot(lat_ref[...], w1bT_ref[...],
                   preferred_element_type=jnp.float32)
         + b1_ref[...])
    h = h * _sigmoid(h)
    out = (jnp.dot(h, w2T_ref[...], preferred_element_type=jnp.float32)
           + b2_ref[...])
    mu_ref[...] = out[:, :LATENT]
    lv_ref[...] = out[:, LATENT:]


def kernel(lattice, fracs, species, batch_indices, emb, fc_w, fc_b,
           bn1_g, bn1_b, bn2_g, bn2_b, w1, b1, w2, b2):
    f32 = jnp.float32
    batch = batch_indices.astype(jnp.int32)
    batchT = batch.reshape(1, N)
    batch_col = batch.reshape(N, 1)
    fracsT8 = jnp.zeros((8, N), f32).at[:3].set(fracs.T)
    lat9 = lattice.reshape(G, 9)
    latT16 = jnp.zeros((16, G), f32).at[:9].set(lat9.T)

    ltri = (jnp.arange(G)[:, None] > jnp.arange(G)[None, :]).astype(f32)
    cartT, rs_col, re_col = pl.pallas_call(
        _cart_kernel,
        out_shape=[jax.ShapeDtypeStruct((8, N), f32),
                   jax.ShapeDtypeStruct((N, 1), jnp.int32),
                   jax.ShapeDtypeStruct((N, 1), jnp.int32)],
    )(fracsT8, batchT, batch_col, latT16, ltri)

    cart_nt = jnp.zeros((N, 4), f32).at[:, :3].set(cartT[:3].T)
    row_s = rs_col.reshape(N)
    row_e = re_col.reshape(N)

    grid_spec = pltpu.PrefetchScalarGridSpec(
        num_scalar_prefetch=2,
        grid=(N // R,),
        in_specs=[
            pl.BlockSpec((8, N), lambda i, *_: (0, 0)),
            pl.BlockSpec((1, N), lambda i, *_: (0, 0)),
            pl.BlockSpec((R, 4), lambda i, *_: (i, 0)),
            pl.BlockSpec((R, 1), lambda i, *_: (i, 0)),
            pl.BlockSpec((R, 1), lambda i, *_: (i, 0)),
            pl.BlockSpec((R, 1), lambda i, *_: (i, 0)),
        ],
        out_specs=[
            pl.BlockSpec((R, 16), lambda i, *_: (i, 0)),
            pl.BlockSpec((R, 16), lambda i, *_: (i, 0)),
        ],
    )
    idx16, dist16 = pl.pallas_call(
        _topk_kernel,
        grid_spec=grid_spec,
        out_shape=[jax.ShapeDtypeStruct((N, 16), f32),
                   jax.ShapeDtypeStruct((N, 16), f32)],
    )(row_s, row_e, cartT, batchT, cart_nt, batch_col, rs_col, re_col)

    src_rm = idx16[:, :K].T.reshape(E).astype(jnp.int32)

    emb_pad = jnp.zeros((emb.shape[0], 128), f32).at[:, :F].set(emb)
    atom = _sc_gather(emb_pad, species.astype(jnp.int32))

    offs = jnp.linspace(0.0, 8.0, NBR).astype(f32).reshape(1, NBR)
    coef = (-0.5 / (offs[0, 1] - offs[0, 0]) ** 2).reshape(1, 1)

    for i in range(NCONV):
        asrc = _sc_gather(atom, src_rm).reshape(K, N, 128)
        fcwT = fc_w[i].T
        fcb = fc_b[i].reshape(1, 2 * F)
        z3, st = pl.pallas_call(
            _convA_kernel,
            grid=(N // BN,),
            in_specs=[
                pl.BlockSpec((K, BN, 128), lambda n: (0, n, 0)),
                pl.BlockSpec((BN, 128), lambda n: (n, 0)),
                pl.BlockSpec((BN, 16), lambda n: (n, 0)),
                pl.BlockSpec((1, NBR), lambda n: (0, 0)),
                pl.BlockSpec((1, 1), lambda n: (0, 0)),
                pl.BlockSpec((3 * F, 2 * F), lambda n: (0, 0)),
                pl.BlockSpec((1, 2 * F), lambda n: (0, 0)),
            ],
            out_specs=[
                pl.BlockSpec((K, BN, 2 * F), lambda n: (0, n, 0)),
                pl.BlockSpec((8, 2 * F), lambda n: (0, 0)),
            ],
            out_shape=[jax.ShapeDtypeStruct((K, N, 2 * F), jnp.bfloat16),
                       jax.ShapeDtypeStruct((8, 2 * F), f32)],
        )(asrc, atom, dist16, offs, coef, fcwT, fcb)

        upd, st2 = pl.pallas_call(
            _convB_kernel,
            grid=(N // BN,),
            in_specs=[
                pl.BlockSpec((K, BN, 2 * F), lambda n: (0, n, 0)),
                pl.BlockSpec((8, 2 * F), lambda n: (0, 0)),
                pl.BlockSpec((1, 2 * F), lambda n: (0, 0)),
                pl.BlockSpec((1, 2 * F), lambda n: (0, 0)),
                pl.BlockSpec((BN, 16), lambda n: (n, 0)),
            ],
            out_specs=[
                pl.BlockSpec((BN, F), lambda n: (n, 0)),
                pl.BlockSpec((8, F), lambda n: (0, 0)),
            ],
            out_shape=[jax.ShapeDtypeStruct((N, F), f32),
                       jax.ShapeDtypeStruct((8, F), f32)],
        )(z3, st, bn1_g[i].reshape(1, 2 * F), bn1_b[i].reshape(1, 2 * F),
          dist16)

        atom = pl.pallas_call(
            _convC_kernel,
            out_shape=jax.ShapeDtypeStruct((N, 128), f32),
        )(atom, upd, st2, bn2_g[i].reshape(1, F), bn2_b[i].reshape(1, F))

    lat16 = jnp.zeros((G, 16), f32).at[:, :9].set(lat9)
    w1aT = w1[:, :F].T
    w1bT = jnp.zeros((16, 128), f32).at[:9].set(w1[:, F:].T)
    mu, lv = pl.pallas_call(
        _final_kernel,
        out_shape=[jax.ShapeDtypeStruct((G, LATENT), f32),
                   jax.ShapeDtypeStruct((G, LATENT), f32)],
    )(atom, batchT, lat16, w1aT, w1bT, b1.reshape(1, 128),
      w2.T, b2.reshape(1, 2 * LATENT))
    return mu, lv


# speedup vs baseline: 1.2397x; 1.2208x over previous
import functools

import jax
import jax.numpy as jnp
from jax import lax
from jax.experimental import pallas as pl
from jax.experimental.pallas import tpu as pltpu
from jax.experimental.pallas import tpu_sc as plsc

N = 4096
G = 64
F = 64
NBR = 64
K = 12
NCONV = 3
LATENT = 128
E = N * K
BIG = 1e12
INF = 1e30
VALID_T = 1e5

R = 128
CCH = 256
BN = 256
EPS_BN = 1e-5

_SC_NC, _SC_NS = 2, 16
_SC_NW = _SC_NC * _SC_NS


def _sc_gather(table, idx):
    V, D = table.shape
    B = idx.shape[0]
    bpw = B // _SC_NW
    ch = min(bpw, 512)
    rounds = bpw // ch
    mesh = plsc.VectorSubcoreMesh(
        core_axis_name="c", subcore_axis_name="s",
        num_cores=_SC_NC, num_subcores=_SC_NS)

    @functools.partial(
        pl.kernel, mesh=mesh,
        out_type=jax.ShapeDtypeStruct((B, D), table.dtype),
        scratch_types=[
            pltpu.VMEM((ch,), jnp.int32),
            pltpu.VMEM((ch, D), table.dtype),
            pltpu.SemaphoreType.DMA,
        ])
    def gk(table_hbm, idx_hbm, out_hbm, idx_v, rows_v, sem):
        wid = lax.axis_index("s") * _SC_NC + lax.axis_index("c")
        base = wid * bpw
        for r in range(rounds):
            b = base + r * ch
            pltpu.sync_copy(idx_hbm.at[pl.ds(b, ch)], idx_v)
            pltpu.async_copy(table_hbm.at[idx_v], rows_v, sem).wait()
            pltpu.sync_copy(rows_v, out_hbm.at[pl.ds(b, ch)])

    return gk(table, idx)


def _cart_kernel(fracsT_ref, batchT_ref, bcol_ref, lat_ref, ltri_ref,
                 out_ref, rs_ref, re_ref):
    bt = batchT_ref[...]
    lat = lat_ref[...]
    oh = (lax.broadcasted_iota(jnp.int32, (G, 1), 0) == bt
          ).astype(jnp.float32)
    acc = jnp.dot(lat, oh, precision=lax.Precision.HIGHEST,
                  preferred_element_type=jnp.float32)
    f0 = fracsT_ref[0:1, :]
    f1 = fracsT_ref[1:2, :]
    f2 = fracsT_ref[2:3, :]
    rows = []
    for d in range(3):
        rows.append(f0 * acc[d:d + 1, :] + f1 * acc[3 + d:4 + d, :]
                    + f2 * acc[6 + d:7 + d, :])
    si = lax.broadcasted_iota(jnp.int32, (8, N), 0)
    out = jnp.where(si == 0, rows[0],
                    jnp.where(si == 1, rows[1],
                              jnp.where(si == 2, rows[2], 0.0)))
    out_ref[...] = out
    bcol = bcol_ref[...]
    ohT = (bcol == lax.broadcasted_iota(jnp.int32, (1, G), 1)
           ).astype(jnp.float32)
    ones = jnp.ones((N, 1), jnp.float32)
    hi = lax.Precision.HIGHEST
    cnt_col = jnp.dot(oh, ones, precision=hi,
                      preferred_element_type=jnp.float32)
    seg_s = jnp.dot(ltri_ref[...], cnt_col, precision=hi,
                    preferred_element_type=jnp.float32)
    seg_e = seg_s + cnt_col
    rs = jnp.dot(ohT, seg_s, precision=hi,
                 preferred_element_type=jnp.float32)
    re = jnp.dot(ohT, seg_e, precision=hi,
                 preferred_element_type=jnp.float32)
    rs_ref[...] = rs.astype(jnp.int32)
    re_ref[...] = re.astype(jnp.int32)


def _topk_kernel(rs_p, re_p, cartT_ref, batchT_ref, cartn_ref, bcol_ref,
                 rs_ref, re_ref, idx_ref, dist_ref):
    pid = pl.program_id(0)
    r0 = pid * R
    lo = rs_p[r0]
    hi = re_p[r0 + R - 1]
    c_lo = lo // CCH
    c_hi = (hi + CCH - 1) // CCH

    xr0 = cartT_ref[0:1, pl.ds(r0, R)]
    xr1 = cartT_ref[1:2, pl.ds(r0, R)]
    xr2 = cartT_ref[2:3, pl.ds(r0, R)]
    brow = batchT_ref[:, pl.ds(r0, R)]
    rowid = r0 + lax.broadcasted_iota(jnp.int32, (1, R), 1)
    CS = 16

    def chunk_body(c, carry):
        cv, ci = carry
        cb = c * CCH
        bc = bcol_ref[pl.ds(cb, CCH), :]
        colid = cb + lax.broadcasted_iota(jnp.int32, (CCH, 1), 0)
        d0 = cartn_ref[pl.ds(cb, CCH), 0:1] - xr0
        d1 = cartn_ref[pl.ds(cb, CCH), 1:2] - xr1
        d2 = cartn_ref[pl.ds(cb, CCH), 2:3] - xr2
        dist = d0 * d0 + d1 * d1 + d2 * d2
        ok = (bc == brow) & (colid != rowid)
        dist = jnp.where(ok, dist, BIG)
        idxf = jnp.broadcast_to(colid.astype(jnp.float32), (CCH, R))
        av = jnp.concatenate([cv, dist], axis=0)
        ai = jnp.concatenate([ci, idxf], axis=0)
        sub = lax.broadcasted_iota(jnp.int32, (CS, R), 0)
        nv = jnp.full((CS, R), INF, jnp.float32)
        ni = jnp.full((CS, R), 1e9, jnp.float32)
        for t in range(K):
            m = jnp.min(av, axis=0, keepdims=True)
            ismin = av == m
            sel = jnp.min(jnp.where(ismin, ai, 1e9), axis=0, keepdims=True)
            hit = ismin & (ai == sel)
            nv = jnp.where(sub == t, m, nv)
            ni = jnp.where(sub == t, sel, ni)
            av = jnp.where(hit, INF, av)
        return nv, ni

    cv0 = jnp.full((CS, R), INF, jnp.float32)
    ci0 = jnp.full((CS, R), 1e9, jnp.float32)
    cv, ci = lax.fori_loop(c_lo, c_hi, chunk_body, (cv0, ci0))

    s = rs_ref[0:1, pl.ds(r0, R)]
    e = re_ref[0:1, pl.ds(r0, R)]
    cnt = e - s - 1
    tl = lax.broadcasted_iota(jnp.int32, (CS, R), 0)
    j = tl - cnt
    fill = jnp.where(j < s, j,
                     jnp.where(j == s, rowid, e + (j - s) - 1))
    use_fill = tl >= cnt
    fidx = jnp.where(use_fill, fill.astype(jnp.float32), ci)
    fval = jnp.where(use_fill, jnp.float32(BIG), cv)
    idx_ref[...] = fidx
    dist_ref[...] = jnp.sqrt(fval + 1e-8)


def _convA_kernel(asrc_ref, atom_ref, dist_ref, offs_ref, coef_ref,
                  w_ref, b_ref, z_ref, st_ref):
    i = pl.program_id(0)
    atom = atom_ref[...][:, :F].astype(jnp.bfloat16)
    w = w_ref[...]
    b = b_ref[...]
    offs = offs_ref[...]
    coef = coef_ref[...]
    asrc = asrc_ref[...][:, :, :F].astype(jnp.bfloat16).reshape(K * BN, F)
    adst = jnp.concatenate([atom] * K, axis=0)
    nbrs = []
    for t in range(K):
        dcol = dist_ref[:, t:t + 1]
        valid = (dcol < VALID_T).astype(jnp.float32)
        dlt = dcol - offs
        nbrs.append((jnp.exp(coef * dlt * dlt) * valid
                     ).astype(jnp.bfloat16))
    nbr = jnp.concatenate(nbrs, axis=0)
    tot = jnp.concatenate([asrc, adst, nbr], axis=1)
    z = jnp.dot(tot, w.astype(jnp.bfloat16),
                preferred_element_type=jnp.float32) + b
    z_ref[...] = z.astype(jnp.bfloat16).reshape(K, BN, 2 * F)
    ssum = jnp.sum(z, axis=0, keepdims=True)
    ssq = jnp.sum(z * z, axis=0, keepdims=True)
    sub = lax.broadcasted_iota(jnp.int32, (8, 2 * F), 0)
    contrib = (jnp.where(sub == 0, ssum, 0.0)
               + jnp.where(sub == 1, ssq, 0.0))

    @pl.when(i == 0)
    def _():
        st_ref[...] = jnp.zeros((8, 2 * F), jnp.float32)

    st_ref[...] += contrib


def _softplus(x):
    return jnp.maximum(x, 0.0) + jnp.log1p(jnp.exp(-jnp.abs(x)))


def _sigmoid(x):
    return 1.0 / (1.0 + jnp.exp(-x))


def _convB_kernel(z_ref, st_ref, g_ref, b_ref, dist_ref, upd_ref, st2_ref):
    i = pl.program_id(0)
    st = st_ref[...]
    mean = st[0:1] * (1.0 / E)
    var = st[1:2] * (1.0 / E) - mean * mean
    mult = g_ref[...] * lax.rsqrt(var + EPS_BN)
    add = b_ref[...] - mean * mult
    upd = jnp.zeros((BN, F), jnp.float32)
    for t in range(K):
        z = z_ref[t].astype(jnp.float32) * mult + add
        filt = z[:, :F]
        core = z[:, F:]
        dcol = dist_ref[:, t:t + 1]
        valid = (dcol < VALID_T).astype(jnp.float32)
        upd = upd + _sigmoid(filt) * _softplus(core) * valid
    upd_ref[...] = upd
    s0 = jnp.sum(upd, axis=0, keepdims=True)
    s1 = jnp.sum(upd * upd, axis=0, keepdims=True)
    sub = lax.broadcasted_iota(jnp.int32, (8, F), 0)
    contrib = (jnp.where(sub == 0, s0, 0.0)
               + jnp.where(sub == 1, s1, 0.0))

    @pl.when(i == 0)
    def _():
        st2_ref[...] = jnp.zeros((8, F), jnp.float32)

    st2_ref[...] += contrib


def _convC_kernel(atom_ref, upd_ref, st2_ref, g_ref, b_ref, out_ref):
    st = st2_ref[...]
    mean = st[0:1] * (1.0 / N)
    var = st[1:2] * (1.0 / N) - mean * mean
    mult = g_ref[...] * lax.rsqrt(var + EPS_BN)
    add = b_ref[...] - mean * mult
    x = atom_ref[...][:, :F] + upd_ref[...] * mult + add
    out_ref[:, :F] = _softplus(x)
    out_ref[:, F:] = jnp.zeros((N, 128 - F), jnp.float32)


def _final_kernel(atom_ref, batchT_ref, lat_ref, w1aT_ref, w1bT_ref,
                  b1_ref, w2T_ref, b2_ref, mu_ref, lv_ref):
    bt = batchT_ref[...]
    gi = lax.broadcasted_iota(jnp.int32, (G, N), 0)
    oh = (gi == bt).astype(jnp.float32)
    crys = jnp.dot(oh, atom_ref[...][:, :F],
                   preferred_element_type=jnp.float32)
    cnt = jnp.sum(oh, axis=1, keepdims=True)
    crys = crys / jnp.maximum(cnt, 1.0)
    h = (jnp.dot(crys, w1aT_ref[...], preferred_element_type=jnp.float32)
         + jnp.dot(lat_ref[...], w1bT_ref[...],
                   preferred_element_type=jnp.float32)
         + b1_ref[...])
    h = h * _sigmoid(h)
    out = (jnp.dot(h, w2T_ref[...], preferred_element_type=jnp.float32)
           + b2_ref[...])
    mu_ref[...] = out[:, :LATENT]
    lv_ref[...] = out[:, LATENT:]


def kernel(lattice, fracs, species, batch_indices, emb, fc_w, fc_b,
           bn1_g, bn1_b, bn2_g, bn2_b, w1, b1, w2, b2):
    f32 = jnp.float32
    batch = batch_indices.astype(jnp.int32)
    batchT = batch.reshape(1, N)
    batch_col = batch.reshape(N, 1)
    fracsT8 = jnp.zeros((8, N), f32).at[:3].set(fracs.T)
    lat9 = lattice.reshape(G, 9)
    latT16 = jnp.zeros((16, G), f32).at[:9].set(lat9.T)

    ltri = (jnp.arange(G)[:, None] > jnp.arange(G)[None, :]).astype(f32)
    cartT, rs_col, re_col = pl.pallas_call(
        _cart_kernel,
        out_shape=[jax.ShapeDtypeStruct((8, N), f32),
                   jax.ShapeDtypeStruct((N, 1), jnp.int32),
                   jax.ShapeDtypeStruct((N, 1), jnp.int32)],
    )(fracsT8, batchT, batch_col, latT16, ltri)

    cart_nt = jnp.zeros((N, 4), f32).at[:, :3].set(cartT[:3].T)
    row_s = rs_col.reshape(N)
    row_e = re_col.reshape(N)

    grid_spec = pltpu.PrefetchScalarGridSpec(
        num_scalar_prefetch=2,
        grid=(N // R,),
        in_specs=[
            pl.BlockSpec((8, N), lambda i, *_: (0, 0)),
            pl.BlockSpec((1, N), lambda i, *_: (0, 0)),
            pl.BlockSpec((N, 4), lambda i, *_: (0, 0)),
            pl.BlockSpec((N, 1), lambda i, *_: (0, 0)),
            pl.BlockSpec((1, N), lambda i, *_: (0, 0)),
            pl.BlockSpec((1, N), lambda i, *_: (0, 0)),
        ],
        out_specs=[
            pl.BlockSpec((16, R), lambda i, *_: (0, i)),
            pl.BlockSpec((16, R), lambda i, *_: (0, i)),
        ],
    )
    idx16t, dist16t = pl.pallas_call(
        _topk_kernel,
        grid_spec=grid_spec,
        out_shape=[jax.ShapeDtypeStruct((16, N), f32),
                   jax.ShapeDtypeStruct((16, N), f32)],
    )(row_s, row_e, cartT, batchT, cart_nt, batch_col,
      rs_col.reshape(1, N), re_col.reshape(1, N))

    src_rm = idx16t[:K].reshape(E).astype(jnp.int32)
    dist16 = dist16t.T

    emb_pad = jnp.zeros((emb.shape[0], 128), f32).at[:, :F].set(emb)
    atom = _sc_gather(emb_pad, species.astype(jnp.int32))

    offs = jnp.linspace(0.0, 8.0, NBR).astype(f32).reshape(1, NBR)
    coef = (-0.5 / (offs[0, 1] - offs[0, 0]) ** 2).reshape(1, 1)

    for i in range(NCONV):
        asrc = _sc_gather(atom, src_rm).reshape(K, N, 128)
        fcwT = fc_w[i].T
        fcb = fc_b[i].reshape(1, 2 * F)
        z3, st = pl.pallas_call(
            _convA_kernel,
            grid=(N // BN,),
            in_specs=[
                pl.BlockSpec((K, BN, 128), lambda n: (0, n, 0)),
                pl.BlockSpec((BN, 128), lambda n: (n, 0)),
                pl.BlockSpec((BN, 16), lambda n: (n, 0)),
                pl.BlockSpec((1, NBR), lambda n: (0, 0)),
                pl.BlockSpec((1, 1), lambda n: (0, 0)),
                pl.BlockSpec((3 * F, 2 * F), lambda n: (0, 0)),
                pl.BlockSpec((1, 2 * F), lambda n: (0, 0)),
            ],
            out_specs=[
                pl.BlockSpec((K, BN, 2 * F), lambda n: (0, n, 0)),
                pl.BlockSpec((8, 2 * F), lambda n: (0, 0)),
            ],
            out_shape=[jax.ShapeDtypeStruct((K, N, 2 * F), jnp.bfloat16),
                       jax.ShapeDtypeStruct((8, 2 * F), f32)],
        )(asrc, atom, dist16, offs, coef, fcwT, fcb)

        upd, st2 = pl.pallas_call(
            _convB_kernel,
            grid=(N // BN,),
            in_specs=[
                pl.BlockSpec((K, BN, 2 * F), lambda n: (0, n, 0)),
                pl.BlockSpec((8, 2 * F), lambda n: (0, 0)),
                pl.BlockSpec((1, 2 * F), lambda n: (0, 0)),
                pl.BlockSpec((1, 2 * F), lambda n: (0, 0)),
                pl.BlockSpec((BN, 16), lambda n: (n, 0)),
            ],
            out_specs=[
                pl.BlockSpec((BN, F), lambda n: (n, 0)),
                pl.BlockSpec((8, F), lambda n: (0, 0)),
            ],
            out_shape=[jax.ShapeDtypeStruct((N, F), f32),
                       jax.ShapeDtypeStruct((8, F), f32)],
        )(z3, st, bn1_g[i].reshape(1, 2 * F), bn1_b[i].reshape(1, 2 * F),
          dist16)

        atom = pl.pallas_call(
            _convC_kernel,
            out_shape=jax.ShapeDtypeStruct((N, 128), f32),
        )(atom, upd, st2, bn2_g[i].reshape(1, F), bn2_b[i].reshape(1, F))

    lat16 = jnp.zeros((G, 16), f32).at[:, :9].set(lat9)
    w1aT = w1[:, :F].T
    w1bT = jnp.zeros((16, 128), f32).at[:9].set(w1[:, F:].T)
    mu, lv = pl.pallas_call(
        _final_kernel,
        out_shape=[jax.ShapeDtypeStruct((G, LATENT), f32),
                   jax.ShapeDtypeStruct((G, LATENT), f32)],
    )(atom, batchT, lat16, w1aT, w1bT, b1.reshape(1, 128),
      w2.T, b2.reshape(1, 2 * LATENT))
    return mu, lv
